# baseline (speedup 1.0000x reference)
import functools

import jax
import jax.numpy as jnp
from jax import lax
from jax.experimental import pallas as pl
from jax.experimental.pallas import tpu as pltpu
from jax.experimental.pallas import tpu_sc as plsc

N = 10000
E = 320000
D_IN = 128
D_H = 128
D_OUT = 64

NC = 2
NS = 16
CHUNK = 128
IB = 4
NB0 = 40
NB1 = 0
DB0 = 20
DB1 = 20
TB = NS * (NB0 + NB1)
EPAD = TB * IB * CHUNK
SINK = N
ROWS_PER_TILE = 640
NSINK = NS * ROWS_PER_TILE

_mesh = plsc.VectorSubcoreMesh(core_axis_name="c", subcore_axis_name="s")


def _tile_blocks(cid, sid, nb0, nb1):
    base = lax.select(cid == 0, sid * nb0, NS * nb0 + sid * nb1)
    nb = lax.select(cid == 0, nb0, nb1)
    return base, nb


@functools.partial(
    pl.kernel,
    mesh=_mesh,
    out_type=jax.ShapeDtypeStruct((NC, NSINK), jnp.float32),
    scratch_types=[
        pltpu.VMEM((2, IB, CHUNK), jnp.int32),
        pltpu.VMEM((CHUNK,), jnp.float32),
        pltpu.VMEM((ROWS_PER_TILE,), jnp.float32),
        pltpu.VMEM_SHARED((NSINK,), jnp.float32),
    ],
)
def _degree_kernel(e_hbm, out_hbm, eb_v, ones_v, zero_v, acc):
    cid = lax.axis_index("c")
    sid = lax.axis_index("s")
    base, nb = _tile_blocks(cid, sid, DB0, DB1)

    for i in range(CHUNK // 16):
        ones_v[pl.ds(i * 16, 16)] = jnp.ones((16,), jnp.float32)

    def _z(i, carry):
        zero_v[pl.ds(i * 16, 16)] = jnp.zeros((16,), jnp.float32)
        return carry

    lax.fori_loop(0, ROWS_PER_TILE // 16, _z, 0)
    pltpu.sync_copy(zero_v, acc.at[pl.ds(sid * ROWS_PER_TILE, ROWS_PER_TILE)])
    plsc.subcore_barrier()

    def _block(blk, carry):
        pltpu.sync_copy(e_hbm.at[base + blk], eb_v)

        def _scatter(c, carry2):
            pltpu.sync_copy(ones_v, acc.at[eb_v.at[1, c]], add=True)
            return carry2

        lax.fori_loop(0, IB, _scatter, 0)
        return carry

    lax.fori_loop(0, nb, _block, 0)
    plsc.subcore_barrier()
    pltpu.sync_copy(
        acc.at[pl.ds(sid * ROWS_PER_TILE, ROWS_PER_TILE)],
        out_hbm.at[cid, pl.ds(sid * ROWS_PER_TILE, ROWS_PER_TILE)],
    )


def _make_row_scatter(D):
    @functools.partial(
        pl.kernel,
        mesh=_mesh,
        out_type=jax.ShapeDtypeStruct((NC, NSINK, D), jnp.float32),
        scratch_types=[
            pltpu.VMEM((2, 2, IB, CHUNK), jnp.int32),
            pltpu.VMEM((2, CHUNK, D), jnp.float32),
            pltpu.VMEM_SHARED((NSINK, D), jnp.float32),
            pltpu.SemaphoreType.DMA,
            pltpu.SemaphoreType.DMA,
        ],
    )
    def _row_scatter(g_hbm, e_hbm, out_hbm, idx_v, rows_v, acc, sem_i, sem_g):
        cid = lax.axis_index("c")
        sid = lax.axis_index("s")
        base, nb = _tile_blocks(cid, sid, NB0, NB1)

        @pl.when(nb > 0)
        def _():
            pltpu.async_copy(e_hbm.at[base], idx_v.at[0], sem_i)

        def _z(i, carry):
            r = i // (D // 16)
            c = lax.rem(i, D // 16)
            rows_v[0, r, pl.ds(c * 16, 16)] = jnp.zeros((16,), jnp.float32)
            return carry

        lax.fori_loop(0, CHUNK * D // 16, _z, 0)
        for b in range(ROWS_PER_TILE // CHUNK):
            pltpu.sync_copy(
                rows_v.at[0],
                acc.at[pl.ds(sid * ROWS_PER_TILE + b * CHUNK, CHUNK)],
            )
        plsc.subcore_barrier()

        def _block(blk, carry):
            pb = lax.rem(blk, 2)
            pltpu.make_async_copy(e_hbm.at[0], idx_v.at[pb], sem_i).wait()

            @pl.when(blk + 1 < nb)
            def _():
                pltpu.async_copy(
                    e_hbm.at[base + blk + 1], idx_v.at[lax.rem(blk + 1, 2)],
                    sem_i)

            src_b = idx_v.at[pb, 0]
            dst_b = idx_v.at[pb, 1]

            pltpu.async_copy(g_hbm.at[src_b.at[0]], rows_v.at[0], sem_g)

            def _chunk(c, carry2):
                @pl.when(c + 1 < IB)
                def _():
                    pltpu.async_copy(
                        g_hbm.at[src_b.at[c + 1]],
                        rows_v.at[lax.rem(c + 1, 2)], sem_g)

                buf = rows_v.at[lax.rem(c, 2)]
                pltpu.make_async_copy(
                    g_hbm.at[pl.ds(0, CHUNK)], buf, sem_g).wait()
                pltpu.sync_copy(buf, acc.at[dst_b.at[c]], add=True)
                return carry2

            lax.fori_loop(0, IB, _chunk, 0)
            return carry

        lax.fori_loop(0, nb, _block, 0)
        plsc.subcore_barrier()
        pltpu.sync_copy(
            acc.at[pl.ds(sid * ROWS_PER_TILE, ROWS_PER_TILE)],
            out_hbm.at[cid, pl.ds(sid * ROWS_PER_TILE, ROWS_PER_TILE)],
        )

    return _row_scatter


_row_scatter_h = _make_row_scatter(D_H)


def _dot(a, b):
    return jnp.dot(a, b, precision=lax.Precision.HIGHEST,
                   preferred_element_type=jnp.float32)


def _tc1_body(x_ref, w1_ref, b1_ref, wc1_ref, d0_ref, d1_ref, g1_ref):
    h = jnp.maximum(_dot(x_ref[...], w1_ref[...]) + b1_ref[...], 0.0)
    dis = lax.rsqrt(d0_ref[...] + d1_ref[...])
    g1_ref[...] = _dot(h, wc1_ref[...]) * dis


def _tc2_body(s0_ref, s1_ref, g1_ref, b1_ref, wc2_ref, d0_ref, d1_ref, g2_ref):
    dis = lax.rsqrt(d0_ref[...] + d1_ref[...])
    h = jnp.maximum(
        dis * (s0_ref[...] + s1_ref[...] + g1_ref[...]) + b1_ref[...], 0.0)
    g2_ref[...] = jnp.concatenate(
        [_dot(h, wc2_ref[...]) * dis, jnp.zeros((N, D_H - D_OUT), jnp.float32)],
        axis=1)


def _tc3_body(s0_ref, s1_ref, g2_ref, b2_ref, d0_ref, d1_ref, out_ref):
    dis = lax.rsqrt(d0_ref[...] + d1_ref[...])
    agg = (s0_ref[...] + s1_ref[...] + g2_ref[...])[:, :D_OUT]
    out_ref[...] = dis * agg + b2_ref[...]


_tc1 = pl.pallas_call(
    _tc1_body, out_shape=jax.ShapeDtypeStruct((N, D_H), jnp.float32))
_tc2 = pl.pallas_call(
    _tc2_body, out_shape=jax.ShapeDtypeStruct((N, D_H), jnp.float32))
_tc3 = pl.pallas_call(
    _tc3_body, out_shape=jax.ShapeDtypeStruct((N, D_OUT), jnp.float32))


def kernel(x, edge_index, fc1_w, fc1_b, conv1_w, conv1_b, conv2_w, conv2_b):
    src = edge_index[0]
    dst = edge_index[1]
    pad = EPAD - E
    src4 = jnp.concatenate(
        [src, jnp.zeros((pad,), jnp.int32)]).reshape(TB, 1, IB, CHUNK)
    dst4 = jnp.concatenate(
        [dst, jnp.full((pad,), SINK, jnp.int32)]).reshape(TB, 1, IB, CHUNK)
    e4 = jnp.concatenate([src4, dst4], axis=1)

    degp = _degree_kernel(e4)
    d0 = degp[0, :N, None] + 1.0
    d1 = degp[1, :N, None]

    g1 = _tc1(x, fc1_w, fc1_b.reshape(1, D_H), conv1_w, d0, d1)
    s1 = _row_scatter_h(g1, e4)
    g2 = _tc2(s1[0, :N], s1[1, :N], g1, conv1_b.reshape(1, D_H), conv2_w,
              d0, d1)
    s2 = _row_scatter_h(g2, e4)
    out = _tc3(s2[0, :N], s2[1, :N], g2, conv2_b.reshape(1, D_OUT), d0, d1)
    return out


# speedup vs baseline: 3.2007x; 3.2007x over previous
import functools

import jax
import jax.numpy as jnp
from jax import lax
from jax.experimental import pallas as pl
from jax.experimental.pallas import tpu as pltpu
from jax.experimental.pallas import tpu_sc as plsc

N = 10000
E = 320000
D_IN = 128
D_H = 128
D_OUT = 64

NC = 2
NS = 16
CHUNK = 128
IB = 4
NB0 = 20
NB1 = 20
DB0 = 20
DB1 = 20
TB = NS * (NB0 + NB1)
EPAD = TB * IB * CHUNK
SINK = N
ROWS_PER_TILE = 640
NSINK = NS * ROWS_PER_TILE

_mesh = plsc.VectorSubcoreMesh(core_axis_name="c", subcore_axis_name="s")


def _tile_blocks(cid, sid, nb0, nb1):
    base = lax.select(cid == 0, sid * nb0, NS * nb0 + sid * nb1)
    nb = lax.select(cid == 0, nb0, nb1)
    return base, nb


@functools.partial(
    pl.kernel,
    mesh=_mesh,
    out_type=jax.ShapeDtypeStruct((NC, NSINK), jnp.float32),
    scratch_types=[
        pltpu.VMEM((2, IB, CHUNK), jnp.int32),
        pltpu.VMEM((CHUNK,), jnp.float32),
        pltpu.VMEM((ROWS_PER_TILE,), jnp.float32),
        pltpu.VMEM_SHARED((NSINK,), jnp.float32),
    ],
)
def _degree_kernel(e_hbm, out_hbm, eb_v, ones_v, zero_v, acc):
    cid = lax.axis_index("c")
    sid = lax.axis_index("s")
    base, nb = _tile_blocks(cid, sid, DB0, DB1)

    for i in range(CHUNK // 16):
        ones_v[pl.ds(i * 16, 16)] = jnp.ones((16,), jnp.float32)

    def _z(i, carry):
        zero_v[pl.ds(i * 16, 16)] = jnp.zeros((16,), jnp.float32)
        return carry

    lax.fori_loop(0, ROWS_PER_TILE // 16, _z, 0)
    pltpu.sync_copy(zero_v, acc.at[pl.ds(sid * ROWS_PER_TILE, ROWS_PER_TILE)])
    plsc.subcore_barrier()

    def _block(blk, carry):
        pltpu.sync_copy(e_hbm.at[base + blk], eb_v)

        def _scatter(c, carry2):
            pltpu.sync_copy(ones_v, acc.at[eb_v.at[1, c]], add=True)
            return carry2

        lax.fori_loop(0, IB, _scatter, 0)
        return carry

    lax.fori_loop(0, nb, _block, 0)
    plsc.subcore_barrier()
    pltpu.sync_copy(
        acc.at[pl.ds(sid * ROWS_PER_TILE, ROWS_PER_TILE)],
        out_hbm.at[cid, pl.ds(sid * ROWS_PER_TILE, ROWS_PER_TILE)],
    )


def _make_row_scatter(D):
    @functools.partial(
        pl.kernel,
        mesh=_mesh,
        out_type=jax.ShapeDtypeStruct((NC, NSINK, D), jnp.float32),
        scratch_types=[
            pltpu.VMEM((2, 2, IB, CHUNK), jnp.int32),
            pltpu.VMEM((2, CHUNK, D), jnp.float32),
            pltpu.VMEM_SHARED((NSINK, D), jnp.float32),
            pltpu.SemaphoreType.DMA,
            pltpu.SemaphoreType.DMA,
        ],
    )
    def _row_scatter(g_hbm, e_hbm, out_hbm, idx_v, rows_v, acc, sem_i, sem_g):
        cid = lax.axis_index("c")
        sid = lax.axis_index("s")
        base, nb = _tile_blocks(cid, sid, NB0, NB1)

        @pl.when(nb > 0)
        def _():
            pltpu.async_copy(e_hbm.at[base], idx_v.at[0], sem_i)

        def _z(i, carry):
            r = i // (D // 16)
            c = lax.rem(i, D // 16)
            rows_v[0, r, pl.ds(c * 16, 16)] = jnp.zeros((16,), jnp.float32)
            return carry

        lax.fori_loop(0, CHUNK * D // 16, _z, 0)
        for b in range(ROWS_PER_TILE // CHUNK):
            pltpu.sync_copy(
                rows_v.at[0],
                acc.at[pl.ds(sid * ROWS_PER_TILE + b * CHUNK, CHUNK)],
            )
        plsc.subcore_barrier()

        def _block(blk, carry):
            pb = lax.rem(blk, 2)
            pltpu.make_async_copy(e_hbm.at[0], idx_v.at[pb], sem_i).wait()

            @pl.when(blk + 1 < nb)
            def _():
                pltpu.async_copy(
                    e_hbm.at[base + blk + 1], idx_v.at[lax.rem(blk + 1, 2)],
                    sem_i)

            src_b = idx_v.at[pb, 0]
            dst_b = idx_v.at[pb, 1]

            pltpu.async_copy(g_hbm.at[src_b.at[0]], rows_v.at[0], sem_g)

            def _chunk(c, carry2):
                @pl.when(c + 1 < IB)
                def _():
                    pltpu.async_copy(
                        g_hbm.at[src_b.at[c + 1]],
                        rows_v.at[lax.rem(c + 1, 2)], sem_g)

                buf = rows_v.at[lax.rem(c, 2)]
                pltpu.make_async_copy(
                    g_hbm.at[pl.ds(0, CHUNK)], buf, sem_g).wait()
                pltpu.sync_copy(buf, acc.at[dst_b.at[c]], add=True)
                return carry2

            lax.fori_loop(0, IB, _chunk, 0)
            return carry

        lax.fori_loop(0, nb, _block, 0)
        plsc.subcore_barrier()
        pltpu.sync_copy(
            acc.at[pl.ds(sid * ROWS_PER_TILE, ROWS_PER_TILE)],
            out_hbm.at[cid, pl.ds(sid * ROWS_PER_TILE, ROWS_PER_TILE)],
        )

    return _row_scatter


_row_scatter_h = _make_row_scatter(D_H)


def _dot(a, b):
    return jnp.dot(a, b, precision=lax.Precision.HIGHEST,
                   preferred_element_type=jnp.float32)


def _tc1_body(x_ref, w1_ref, b1_ref, wc1_ref, d0_ref, d1_ref, g1_ref):
    h = jnp.maximum(_dot(x_ref[...], w1_ref[...]) + b1_ref[...], 0.0)
    dis = lax.rsqrt(d0_ref[...] + d1_ref[...])
    g1_ref[...] = _dot(h, wc1_ref[...]) * dis


def _tc2_body(s0_ref, s1_ref, g1_ref, b1_ref, wc2_ref, d0_ref, d1_ref, g2_ref):
    dis = lax.rsqrt(d0_ref[...] + d1_ref[...])
    h = jnp.maximum(
        dis * (s0_ref[...] + s1_ref[...] + g1_ref[...]) + b1_ref[...], 0.0)
    g2_ref[...] = jnp.concatenate(
        [_dot(h, wc2_ref[...]) * dis, jnp.zeros((N, D_H - D_OUT), jnp.float32)],
        axis=1)


def _tc3_body(s0_ref, s1_ref, g2_ref, b2_ref, d0_ref, d1_ref, out_ref):
    dis = lax.rsqrt(d0_ref[...] + d1_ref[...])
    agg = (s0_ref[...] + s1_ref[...] + g2_ref[...])[:, :D_OUT]
    out_ref[...] = dis * agg + b2_ref[...]


_tc1 = pl.pallas_call(
    _tc1_body, out_shape=jax.ShapeDtypeStruct((N, D_H), jnp.float32))
_tc2 = pl.pallas_call(
    _tc2_body, out_shape=jax.ShapeDtypeStruct((N, D_H), jnp.float32))
_tc3 = pl.pallas_call(
    _tc3_body, out_shape=jax.ShapeDtypeStruct((N, D_OUT), jnp.float32))


def kernel(x, edge_index, fc1_w, fc1_b, conv1_w, conv1_b, conv2_w, conv2_b):
    src = edge_index[0]
    dst = edge_index[1]
    pad = EPAD - E
    pad_i = jnp.arange(pad, dtype=jnp.int32)
    src4 = jnp.concatenate(
        [src, pad_i % N]).reshape(TB, 1, IB, CHUNK)
    dst4 = jnp.concatenate(
        [dst, SINK + pad_i % (NSINK - N)]).reshape(TB, 1, IB, CHUNK)
    e4 = jnp.concatenate([src4, dst4], axis=1)

    degp = _degree_kernel(e4)
    d0 = degp[0, :N, None] + 1.0
    d1 = degp[1, :N, None]

    g1 = _tc1(x, fc1_w, fc1_b.reshape(1, D_H), conv1_w, d0, d1)
    s1 = _row_scatter_h(g1, e4)
    g2 = _tc2(s1[0, :N], s1[1, :N], g1, conv1_b.reshape(1, D_H), conv2_w,
              d0, d1)
    s2 = _row_scatter_h(g2, e4)
    out = _tc3(s2[0, :N], s2[1, :N], g2, conv2_b.reshape(1, D_OUT), d0, d1)
    return out


# speedup vs baseline: 3.6351x; 1.1357x over previous
import functools

import jax
import jax.numpy as jnp
from jax import lax
from jax.experimental import pallas as pl
from jax.experimental.pallas import tpu as pltpu
from jax.experimental.pallas import tpu_sc as plsc

N = 10000
E = 320000
D_IN = 128
D_H = 128
D_OUT = 64

NC = 2
NS = 16
CHUNK = 128
IB = 4
NB0 = 20
NB1 = 20
DB0 = 20
DB1 = 20
TB = NS * (NB0 + NB1)
EPAD = TB * IB * CHUNK
SINK = N
ROWS_PER_TILE = 640
NSINK = NS * ROWS_PER_TILE

_mesh = plsc.VectorSubcoreMesh(core_axis_name="c", subcore_axis_name="s")


def _tile_blocks(cid, sid, nb0, nb1):
    base = lax.select(cid == 0, sid * nb0, NS * nb0 + sid * nb1)
    nb = lax.select(cid == 0, nb0, nb1)
    return base, nb


@functools.partial(
    pl.kernel,
    mesh=_mesh,
    out_type=jax.ShapeDtypeStruct((NC, NSINK), jnp.float32),
    scratch_types=[
        pltpu.VMEM((2, IB, CHUNK), jnp.int32),
        pltpu.VMEM((CHUNK,), jnp.float32),
        pltpu.VMEM((ROWS_PER_TILE,), jnp.float32),
        pltpu.VMEM_SHARED((NSINK,), jnp.float32),
    ],
)
def _degree_kernel(e_hbm, out_hbm, eb_v, ones_v, zero_v, acc):
    cid = lax.axis_index("c")
    sid = lax.axis_index("s")
    base, nb = _tile_blocks(cid, sid, DB0, DB1)

    for i in range(CHUNK // 16):
        ones_v[pl.ds(i * 16, 16)] = jnp.ones((16,), jnp.float32)

    def _z(i, carry):
        zero_v[pl.ds(i * 16, 16)] = jnp.zeros((16,), jnp.float32)
        return carry

    lax.fori_loop(0, ROWS_PER_TILE // 16, _z, 0)
    pltpu.sync_copy(zero_v, acc.at[pl.ds(sid * ROWS_PER_TILE, ROWS_PER_TILE)])
    plsc.subcore_barrier()

    def _block(blk, carry):
        pltpu.sync_copy(e_hbm.at[base + blk], eb_v)

        def _scatter(c, carry2):
            pltpu.sync_copy(ones_v, acc.at[eb_v.at[1, c]], add=True)
            return carry2

        lax.fori_loop(0, IB, _scatter, 0)
        return carry

    lax.fori_loop(0, nb, _block, 0)
    plsc.subcore_barrier()
    pltpu.sync_copy(
        acc.at[pl.ds(sid * ROWS_PER_TILE, ROWS_PER_TILE)],
        out_hbm.at[cid, pl.ds(sid * ROWS_PER_TILE, ROWS_PER_TILE)],
    )


def _make_row_scatter(D):
    @functools.partial(
        pl.kernel,
        mesh=_mesh,
        out_type=jax.ShapeDtypeStruct((NC, NSINK, D), jnp.float32),
        scratch_types=[
            pltpu.VMEM((2, 2, IB, CHUNK), jnp.int32),
            pltpu.VMEM((2, CHUNK, D), jnp.float32),
            pltpu.VMEM_SHARED((NSINK, D), jnp.float32),
            pltpu.SemaphoreType.DMA,
            pltpu.SemaphoreType.DMA,
        ],
    )
    def _row_scatter(g_hbm, e_hbm, out_hbm, idx_v, rows_v, acc, sem_i, sem_g):
        cid = lax.axis_index("c")
        sid = lax.axis_index("s")
        base, nb = _tile_blocks(cid, sid, NB0, NB1)

        nchunks = nb * IB

        @pl.when(nb > 0)
        def _():
            pltpu.sync_copy(e_hbm.at[base], idx_v.at[0])

        @pl.when(nb > 1)
        def _():
            pltpu.async_copy(e_hbm.at[base + 1], idx_v.at[1], sem_i)

        def _z(i, carry):
            r = i // (D // 16)
            c = lax.rem(i, D // 16)
            rows_v[0, r, pl.ds(c * 16, 16)] = jnp.zeros((16,), jnp.float32)
            return carry

        lax.fori_loop(0, CHUNK * D // 16, _z, 0)
        for b in range(ROWS_PER_TILE // CHUNK):
            pltpu.sync_copy(
                rows_v.at[0],
                acc.at[pl.ds(sid * ROWS_PER_TILE + b * CHUNK, CHUNK)],
            )
        plsc.subcore_barrier()

        @pl.when(nb > 0)
        def _():
            pltpu.async_copy(
                g_hbm.at[idx_v.at[0, 0, 0]], rows_v.at[0], sem_g)

        def _chunk(j, carry):
            blk = j // IB
            c = lax.rem(j, IB)
            nxt = j + 1
            nblk = nxt // IB
            ncc = lax.rem(nxt, IB)

            @pl.when(jnp.logical_and(c == IB - 1, nxt < nchunks))
            def _():
                pltpu.make_async_copy(
                    e_hbm.at[0], idx_v.at[lax.rem(nblk, 2)], sem_i).wait()

            @pl.when(nxt < nchunks)
            def _():
                pltpu.async_copy(
                    g_hbm.at[idx_v.at[lax.rem(nblk, 2), 0, ncc]],
                    rows_v.at[lax.rem(nxt, 2)], sem_g)

            buf = rows_v.at[lax.rem(j, 2)]
            pltpu.make_async_copy(
                g_hbm.at[pl.ds(0, CHUNK)], buf, sem_g).wait()
            pltpu.sync_copy(
                buf, acc.at[idx_v.at[lax.rem(blk, 2), 1, c]], add=True)

            @pl.when(jnp.logical_and(c == IB - 1, blk + 2 < nb))
            def _():
                pltpu.async_copy(
                    e_hbm.at[base + blk + 2], idx_v.at[lax.rem(blk, 2)],
                    sem_i)

            return carry

        lax.fori_loop(0, nchunks, _chunk, 0)
        plsc.subcore_barrier()
        pltpu.sync_copy(
            acc.at[pl.ds(sid * ROWS_PER_TILE, ROWS_PER_TILE)],
            out_hbm.at[cid, pl.ds(sid * ROWS_PER_TILE, ROWS_PER_TILE)],
        )

    return _row_scatter


_row_scatter_h = _make_row_scatter(D_H)


def _dot(a, b):
    return jnp.dot(a, b, preferred_element_type=jnp.float32)


def _tc1_body(x_ref, w1_ref, b1_ref, wc1_ref, d0_ref, d1_ref, g1_ref):
    h = jnp.maximum(_dot(x_ref[...], w1_ref[...]) + b1_ref[...], 0.0)
    dis = lax.rsqrt(d0_ref[...] + d1_ref[...])
    g1_ref[...] = _dot(h, wc1_ref[...]) * dis


def _tc2_body(s0_ref, s1_ref, g1_ref, b1_ref, wc2_ref, d0_ref, d1_ref, g2_ref):
    dis = lax.rsqrt(d0_ref[...] + d1_ref[...])
    h = jnp.maximum(
        dis * (s0_ref[...] + s1_ref[...] + g1_ref[...]) + b1_ref[...], 0.0)
    g2_ref[...] = jnp.concatenate(
        [_dot(h, wc2_ref[...]) * dis, jnp.zeros((N, D_H - D_OUT), jnp.float32)],
        axis=1)


def _tc3_body(s0_ref, s1_ref, g2_ref, b2_ref, d0_ref, d1_ref, out_ref):
    dis = lax.rsqrt(d0_ref[...] + d1_ref[...])
    agg = (s0_ref[...] + s1_ref[...] + g2_ref[...])[:, :D_OUT]
    out_ref[...] = dis * agg + b2_ref[...]


_tc1 = pl.pallas_call(
    _tc1_body, out_shape=jax.ShapeDtypeStruct((N, D_H), jnp.float32))
_tc2 = pl.pallas_call(
    _tc2_body, out_shape=jax.ShapeDtypeStruct((N, D_H), jnp.float32))
_tc3 = pl.pallas_call(
    _tc3_body, out_shape=jax.ShapeDtypeStruct((N, D_OUT), jnp.float32))


def kernel(x, edge_index, fc1_w, fc1_b, conv1_w, conv1_b, conv2_w, conv2_b):
    src = edge_index[0]
    dst = edge_index[1]
    pad = EPAD - E
    pad_i = jnp.arange(pad, dtype=jnp.int32)
    src4 = jnp.concatenate(
        [src, pad_i % N]).reshape(TB, 1, IB, CHUNK)
    dst4 = jnp.concatenate(
        [dst, SINK + pad_i % (NSINK - N)]).reshape(TB, 1, IB, CHUNK)
    e4 = jnp.concatenate([src4, dst4], axis=1)

    degp = _degree_kernel(e4)
    d0 = degp[0, :N, None] + 1.0
    d1 = degp[1, :N, None]

    g1 = _tc1(x, fc1_w, fc1_b.reshape(1, D_H), conv1_w, d0, d1)
    s1 = _row_scatter_h(g1, e4)
    g2 = _tc2(s1[0, :N], s1[1, :N], g1, conv1_b.reshape(1, D_H), conv2_w,
              d0, d1)
    s2 = _row_scatter_h(g2, e4)
    out = _tc3(s2[0, :N], s2[1, :N], g2, conv2_b.reshape(1, D_OUT), d0, d1)
    return out
